# baseline (speedup 1.0000x reference)
import functools
import math

import jax
import jax.numpy as jnp
from jax import lax
from jax.experimental import pallas as pl
from jax.experimental.pallas import tpu as pltpu
from jax.experimental.pallas import tpu_sc as plsc

_D = 64
_V = 1000000
_SEQ = 200
_BATCH = 4096
_NC = 2
_NS = 16
_NW = _NC * _NS
_C = _BATCH // _NW
_SCALE = math.sqrt(_D)

_NBLK = _V // _C
_FULL_PER_W = _NBLK // _NW
_TAIL_W = _NBLK - _FULL_PER_W * _NW
_PAD = 2 * _D + 2


def _relayout_body(src_hbm, tail_hbm, out_hbm, in_v, out_v, sem_i0, sem_i1,
                   sem_o0, sem_o1):
    wid = lax.axis_index("c") * _NS + lax.axis_index("s")
    sem_i = (sem_i0, sem_i1)
    sem_o = (sem_o0, sem_o1)

    lane = lax.iota(jnp.int32, 16)
    rvec = [lax.shift_right_logical(lane + 16 * tb, 1) for tb in range(8)]
    cvec = [lax.bitwise_and(lane + 16 * tb, 1) * _D for tb in range(8)]

    def transpose_block(bb):
        iv = in_v.at[bb]
        ov = out_v.at[bb]

        @plsc.parallel_loop(0, _D, step=1, unroll=8)
        def _(d):
            df = jnp.full((16,), d, jnp.int32)
            for tb in range(8):
                v = iv[d, pl.ds(16 * tb, 16)] * _SCALE
                plsc.store_scatter(ov, [rvec[tb], cvec[tb] + df], v)

    def in_slice(blk):
        off = pl.multiple_of(blk * _C, _C)
        return src_hbm.at[:, pl.ds(off, _C)]

    def out_slice(blk):
        r0 = pl.multiple_of(blk * (_C // 2), _C // 2)
        return out_hbm.at[pl.ds(r0, _C // 2), :]

    def start_in(blk, bb):
        pltpu.async_copy(in_slice(blk), in_v.at[bb], sem_i[bb])

    def wait_in(blk, bb):
        pltpu.make_async_copy(in_slice(blk), in_v.at[bb], sem_i[bb]).wait()

    def start_out(blk, bb):
        pltpu.async_copy(out_v.at[bb], out_slice(blk), sem_o[bb])

    def wait_out(blk, bb):
        pltpu.make_async_copy(out_v.at[bb], out_slice(blk), sem_o[bb]).wait()

    start_in(wid, 0)

    def pair_body(i2, carry):
        for bb in range(2):
            i = 2 * i2 + bb
            blk = wid + _NW * i

            @pl.when(i + 1 < _FULL_PER_W)
            def _():
                start_in(wid + _NW * (i + 1), 1 - bb)

            wait_in(blk, bb)

            @pl.when(i2 > 0)
            def _():
                wait_out(wid + _NW * (i - 2), bb)

            transpose_block(bb)
            start_out(blk, bb)
        return carry

    lax.fori_loop(0, _FULL_PER_W // 2, pair_body, 0)
    for bb in range(2):
        wait_out(wid + _NW * (_FULL_PER_W - 2 + bb), bb)

    @pl.when(wid < _TAIL_W)
    def _():
        blk = _FULL_PER_W * _NW + wid
        start_in(blk, 0)
        wait_in(blk, 0)
        transpose_block(0)
        start_out(blk, 0)
        wait_out(blk, 0)

    @pl.when(wid == _TAIL_W)
    def _():
        n2 = (_V - _NBLK * _C) // 2
        bounce = out_v.at[0, pl.ds(0, n2), :]
        pltpu.async_copy(tail_hbm, bounce, sem_i0)
        pltpu.make_async_copy(tail_hbm, bounce, sem_i0).wait()
        dst = out_hbm.at[pl.ds(_NBLK * (_C // 2), n2), :]
        pltpu.async_copy(bounce, dst, sem_o0)
        pltpu.make_async_copy(bounce, dst, sem_o0).wait()


def _gather_body(idx_hbm, pe_hbm, table_hbm, out_hbm,
                 idx_v, pe_v, rows_v, trans_v,
                 sem_g0, sem_g1, sem_o0, sem_o1):
    wid = lax.axis_index("c") * _NS + lax.axis_index("s")
    sem_g = (sem_g0, sem_g1)
    sem_o = (sem_o0, sem_o1)

    pltpu.sync_copy(idx_hbm.at[wid], idx_v)
    pltpu.sync_copy(pe_hbm, pe_v)

    lane = lax.iota(jnp.int32, 16)
    dhi = [lax.shift_right_logical(lane + 16 * j, 3) for j in range(_D // 16)]
    dlo = [lax.bitwise_and(lane + 16 * j, 7) for j in range(_D // 16)]

    pltpu.async_copy(table_hbm.at[idx_v.at[0]], rows_v.at[0], sem_g[0])

    def pair_body(s2, carry):
        for b in range(2):
            s = 2 * s2 + b

            @pl.when(s + 1 < _SEQ)
            def _():
                pltpu.async_copy(table_hbm.at[idx_v.at[s + 1]],
                                 rows_v.at[1 - b], sem_g[1 - b])

            pltpu.make_async_copy(table_hbm.at[idx_v.at[s]],
                                  rows_v.at[b], sem_g[b]).wait()

            @pl.when(s2 > 0)
            def _():
                pltpu.make_async_copy(trans_v.at[b, :, :, pl.ds(0, _C)],
                                      out_hbm.at[s - 2, :, wid],
                                      sem_o[b]).wait()

            rv = rows_v.at[b]
            tv = trans_v.at[b]
            pe_s = [pe_v[s, pl.ds(16 * j, 16)] for j in range(_D // 16)]

            @plsc.parallel_loop(0, _C, step=1, unroll=8)
            def _(t):
                col = jnp.full((16,), t, jnp.int32)
                for j in range(_D // 16):
                    v = rv[t, pl.ds(16 * j, 16)] + pe_s[j]
                    plsc.store_scatter(tv, [dhi[j], dlo[j], col], v)

            pltpu.async_copy(tv.at[:, :, pl.ds(0, _C)],
                             out_hbm.at[s, :, wid], sem_o[b])
        return carry

    lax.fori_loop(0, _SEQ // 2, pair_body, 0)
    for b in range(2):
        pltpu.make_async_copy(trans_v.at[b, :, :, pl.ds(0, _C)],
                              out_hbm.at[_SEQ - 2 + b, :, wid],
                              sem_o[b]).wait()


def kernel(x, table, pe):
    mesh = plsc.VectorSubcoreMesh(core_axis_name="c", subcore_axis_name="s")

    relayout = functools.partial(
        pl.kernel,
        mesh=mesh,
        out_type=jax.ShapeDtypeStruct((_V // 2, 2 * _D), jnp.float32),
        scratch_types=[
            pltpu.VMEM((2, _D, _C), jnp.float32),
            pltpu.VMEM((2, _C // 2, 2 * _D), jnp.float32),
            pltpu.SemaphoreType.DMA,
            pltpu.SemaphoreType.DMA,
            pltpu.SemaphoreType.DMA,
            pltpu.SemaphoreType.DMA,
        ],
        compiler_params=pltpu.CompilerParams(use_tc_tiling_on_sc=True,
                                             needs_layout_passes=False),
    )(_relayout_body)

    gather = functools.partial(
        pl.kernel,
        mesh=mesh,
        out_type=jax.ShapeDtypeStruct((_SEQ, _D // 8, _NW, 8, _C),
                                      jnp.float32),
        scratch_types=[
            pltpu.VMEM((_SEQ, _C), jnp.int32),
            pltpu.VMEM((_SEQ, _D), jnp.float32),
            pltpu.VMEM((2, _C, _D), jnp.float32),
            pltpu.VMEM((2, _D // 8, 8, _C + 1), jnp.float32),
            pltpu.SemaphoreType.DMA,
            pltpu.SemaphoreType.DMA,
            pltpu.SemaphoreType.DMA,
            pltpu.SemaphoreType.DMA,
        ],
        compiler_params=pltpu.CompilerParams(use_tc_tiling_on_sc=False,
                                             needs_layout_passes=False),
    )(_gather_body)

    tail = (table[_NBLK * _C:, :] * _SCALE).reshape(-1, 2 * _D)
    scaled = relayout(table.T, tail)
    tbl = scaled.reshape(_V, _D)

    idx = x.astype(jnp.int32).reshape(_NW, _C, _SEQ).transpose(0, 2, 1)
    pe2 = pe[0, :_SEQ, :]
    out5 = gather(idx, pe2, tbl)
    return out5.transpose(2, 4, 0, 1, 3).reshape(_BATCH, _SEQ, _D)


# speedup vs baseline: 1.3523x; 1.3523x over previous
import functools
import math

import jax
import jax.numpy as jnp
from jax import lax
from jax.experimental import pallas as pl
from jax.experimental.pallas import tpu as pltpu
from jax.experimental.pallas import tpu_sc as plsc

_D = 64
_V = 1000000
_SEQ = 200
_BATCH = 4096
_NC = 2
_NS = 16
_NW = _NC * _NS
_C = _BATCH // _NW
_SCALE = math.sqrt(_D)

_R = 256
_DBLK = _V // _R
_DFULL = _DBLK // _NW
_DEXTRA = _DBLK - _DFULL * _NW
_TAIL_ROWS = _V - _DBLK * _R


def _detile_body(src_hbm, out_hbm, in_v, out_v, sem_i0, sem_i1,
                 sem_o0, sem_o1):
    wid = lax.axis_index("c") * _NS + lax.axis_index("s")
    sem_i = (sem_i0, sem_i1)
    sem_o = (sem_o0, sem_o1)

    def in_slice(blk, n):
        off = pl.multiple_of(blk * _R, 8)
        return src_hbm.at[pl.ds(off, n), :]

    def out_slice(blk, n):
        off = pl.multiple_of(blk * (_R // 2), 8)
        return out_hbm.at[pl.ds(off, n // 2), :]

    def start_in(blk, bb, n=_R):
        pltpu.async_copy(in_slice(blk, n), in_v.at[bb, pl.ds(0, n), :],
                         sem_i[bb])

    def wait_in(blk, bb, n=_R):
        pltpu.make_async_copy(in_slice(blk, n), in_v.at[bb, pl.ds(0, n), :],
                              sem_i[bb]).wait()

    def start_out(blk, bb, n=_R):
        pltpu.async_copy(out_v.at[bb, pl.ds(0, n // 2), :], out_slice(blk, n),
                         sem_o[bb])

    def wait_out(blk, bb, n=_R):
        pltpu.make_async_copy(out_v.at[bb, pl.ds(0, n // 2), :],
                              out_slice(blk, n), sem_o[bb]).wait()

    def compact(bb, n=_R):
        iv = in_v.at[bb]
        ov = out_v.at[bb]

        @plsc.parallel_loop(0, n // 2, step=1, unroll=4)
        def _(k):
            for j in range(8):
                ov[k, pl.ds(16 * j, 16)] = iv[2 * k + (j >= 4),
                                              pl.ds(16 * (j % 4), 16)]

    start_in(wid, 0)

    def pair_body(i2, carry):
        for bb in range(2):
            i = 2 * i2 + bb
            blk = wid + _NW * i

            @pl.when(i + 1 < _DFULL)
            def _():
                start_in(wid + _NW * (i + 1), 1 - bb)

            wait_in(blk, bb)

            @pl.when(i2 > 0)
            def _():
                wait_out(wid + _NW * (i - 2), bb)

            compact(bb)
            start_out(blk, bb)
        return carry

    lax.fori_loop(0, _DFULL // 2, pair_body, 0)
    for bb in range(2):
        wait_out(wid + _NW * (_DFULL - 2 + bb), bb)

    @pl.when(wid < _DEXTRA)
    def _():
        blk = _DFULL * _NW + wid
        start_in(blk, 0)
        wait_in(blk, 0)
        compact(0)
        start_out(blk, 0)
        wait_out(blk, 0)

    @pl.when(wid == _DEXTRA)
    def _():
        blk = _DBLK
        start_in(blk, 0, _TAIL_ROWS)
        wait_in(blk, 0, _TAIL_ROWS)
        compact(0, _TAIL_ROWS)
        start_out(blk, 0, _TAIL_ROWS)
        wait_out(blk, 0, _TAIL_ROWS)


def _gather_body(idx_hbm, pe_hbm, table_hbm, out_hbm,
                 idx_v, pe_v, rows_v, trans_v,
                 sem_g0, sem_g1, sem_o0, sem_o1):
    wid = lax.axis_index("c") * _NS + lax.axis_index("s")
    sem_g = (sem_g0, sem_g1)
    sem_o = (sem_o0, sem_o1)

    pltpu.sync_copy(idx_hbm.at[wid], idx_v)
    pltpu.sync_copy(pe_hbm, pe_v)

    lane = lax.iota(jnp.int32, 16)
    dhi = [lax.shift_right_logical(lane + 16 * j, 3) for j in range(_D // 16)]
    dlo = [lax.bitwise_and(lane + 16 * j, 7) for j in range(_D // 16)]

    pltpu.async_copy(table_hbm.at[idx_v.at[0]], rows_v.at[0], sem_g[0])

    def pair_body(s2, carry):
        for b in range(2):
            s = 2 * s2 + b

            @pl.when(s + 1 < _SEQ)
            def _():
                pltpu.async_copy(table_hbm.at[idx_v.at[s + 1]],
                                 rows_v.at[1 - b], sem_g[1 - b])

            pltpu.make_async_copy(table_hbm.at[idx_v.at[s]],
                                  rows_v.at[b], sem_g[b]).wait()

            @pl.when(s2 > 0)
            def _():
                pltpu.make_async_copy(trans_v.at[b, :, :, pl.ds(0, _C)],
                                      out_hbm.at[s - 2, :, wid],
                                      sem_o[b]).wait()

            rv = rows_v.at[b]
            tv = trans_v.at[b]
            pe_s = [pe_v[s, pl.ds(16 * j, 16)] for j in range(_D // 16)]

            @plsc.parallel_loop(0, _C, step=1, unroll=8)
            def _(t):
                col = jnp.full((16,), t, jnp.int32)
                for j in range(_D // 16):
                    v = rv[t, pl.ds(16 * j, 16)] * _SCALE + pe_s[j]
                    plsc.store_scatter(tv, [dhi[j], dlo[j], col], v)

            pltpu.async_copy(tv.at[:, :, pl.ds(0, _C)],
                             out_hbm.at[s, :, wid], sem_o[b])
        return carry

    lax.fori_loop(0, _SEQ // 2, pair_body, 0)
    for b in range(2):
        pltpu.make_async_copy(trans_v.at[b, :, :, pl.ds(0, _C)],
                              out_hbm.at[_SEQ - 2 + b, :, wid],
                              sem_o[b]).wait()


def kernel(x, table, pe):
    mesh = plsc.VectorSubcoreMesh(core_axis_name="c", subcore_axis_name="s")

    detile = functools.partial(
        pl.kernel,
        mesh=mesh,
        out_type=jax.ShapeDtypeStruct((_V // 2, 2 * _D), jnp.float32),
        scratch_types=[
            pltpu.VMEM((2, _R, _D), jnp.float32),
            pltpu.VMEM((2, _R // 2, 2 * _D), jnp.float32),
            pltpu.SemaphoreType.DMA,
            pltpu.SemaphoreType.DMA,
            pltpu.SemaphoreType.DMA,
            pltpu.SemaphoreType.DMA,
        ],
        compiler_params=pltpu.CompilerParams(use_tc_tiling_on_sc=True,
                                             needs_layout_passes=False),
    )(_detile_body)

    gather = functools.partial(
        pl.kernel,
        mesh=mesh,
        out_type=jax.ShapeDtypeStruct((_SEQ, _D // 8, _NW, 8, _C),
                                      jnp.float32),
        scratch_types=[
            pltpu.VMEM((_SEQ, _C), jnp.int32),
            pltpu.VMEM((_SEQ, _D), jnp.float32),
            pltpu.VMEM((2, _C, _D), jnp.float32),
            pltpu.VMEM((2, _D // 8, 8, _C + 1), jnp.float32),
            pltpu.SemaphoreType.DMA,
            pltpu.SemaphoreType.DMA,
            pltpu.SemaphoreType.DMA,
            pltpu.SemaphoreType.DMA,
        ],
        compiler_params=pltpu.CompilerParams(use_tc_tiling_on_sc=False,
                                             needs_layout_passes=False),
    )(_gather_body)

    tbl = detile(table).reshape(_V, _D)

    idx = x.astype(jnp.int32).reshape(_NW, _C, _SEQ).transpose(0, 2, 1)
    pe2 = pe[0, :_SEQ, :]
    out5 = gather(idx, pe2, tbl)
    return out5.transpose(2, 4, 0, 1, 3).reshape(_BATCH, _SEQ, _D)


# speedup vs baseline: 1.3573x; 1.0037x over previous
import functools
import math

import jax
import jax.numpy as jnp
from jax import lax
from jax.experimental import pallas as pl
from jax.experimental.pallas import tpu as pltpu
from jax.experimental.pallas import tpu_sc as plsc

_D = 64
_SEQ = 200
_BATCH = 4096
_NC = 2
_NS = 16
_NW = _NC * _NS
_C = _BATCH // _NW
_SCALE = math.sqrt(_D)


def _sc_embed(idx_hbm, pe_hbm, table_hbm, out_hbm,
              idx_v, pe_v, rows_v, trans_v,
              sem_g0, sem_g1, sem_o0, sem_o1):
    wid = lax.axis_index("c") * _NS + lax.axis_index("s")
    sem_g = (sem_g0, sem_g1)
    sem_o = (sem_o0, sem_o1)

    pltpu.sync_copy(idx_hbm.at[wid], idx_v)
    pltpu.sync_copy(pe_hbm, pe_v)

    lane = lax.iota(jnp.int32, 16)
    dhi = [lax.shift_right_logical(lane + 16 * j, 3) for j in range(_D // 16)]
    dlo = [lax.bitwise_and(lane + 16 * j, 7) for j in range(_D // 16)]

    pltpu.async_copy(table_hbm.at[idx_v.at[0]], rows_v.at[0], sem_g[0])

    def pair_body(s2, carry):
        for b in range(2):
            s = 2 * s2 + b

            @pl.when(s + 1 < _SEQ)
            def _():
                pltpu.async_copy(table_hbm.at[idx_v.at[s + 1]],
                                 rows_v.at[1 - b], sem_g[1 - b])

            pltpu.make_async_copy(table_hbm.at[idx_v.at[s]],
                                  rows_v.at[b], sem_g[b]).wait()

            @pl.when(s2 > 0)
            def _():
                pltpu.make_async_copy(trans_v.at[b, :, :, pl.ds(0, _C)],
                                      out_hbm.at[s - 2, :, wid],
                                      sem_o[b]).wait()

            rv = rows_v.at[b]
            tv = trans_v.at[b]

            pe_s = [pe_v[s, pl.ds(16 * j, 16)] for j in range(_D // 16)]

            @plsc.parallel_loop(0, _C, step=1, unroll=8)
            def _(t):
                col = jnp.full((16,), t, jnp.int32)
                for j in range(_D // 16):
                    v = rv[t, pl.ds(16 * j, 16)] * _SCALE + pe_s[j]
                    plsc.store_scatter(tv, [dhi[j], dlo[j], col], v)

            pltpu.async_copy(tv.at[:, :, pl.ds(0, _C)],
                             out_hbm.at[s, :, wid], sem_o[b])
        return carry

    lax.fori_loop(0, _SEQ // 2, pair_body, 0)

    for b in range(2):
        pltpu.make_async_copy(trans_v.at[b, :, :, pl.ds(0, _C)],
                              out_hbm.at[_SEQ - 2 + b, :, wid],
                              sem_o[b]).wait()


def kernel(x, table, pe):
    mesh = plsc.VectorSubcoreMesh(core_axis_name="c", subcore_axis_name="s")
    fn = functools.partial(
        pl.kernel,
        mesh=mesh,
        out_type=jax.ShapeDtypeStruct((_SEQ, _D // 8, _NW, 8, _C),
                                      jnp.float32),
        scratch_types=[
            pltpu.VMEM((_SEQ, _C), jnp.int32),
            pltpu.VMEM((_SEQ, _D), jnp.float32),
            pltpu.VMEM((2, _C, _D), jnp.float32),
            pltpu.VMEM((2, _D // 8, 8, _C + 1), jnp.float32),
            pltpu.SemaphoreType.DMA,
            pltpu.SemaphoreType.DMA,
            pltpu.SemaphoreType.DMA,
            pltpu.SemaphoreType.DMA,
        ],
        compiler_params=pltpu.CompilerParams(use_tc_tiling_on_sc=False,
                                             needs_layout_passes=False),
    )(_sc_embed)

    idx = x.astype(jnp.int32).reshape(_NW, _C, _SEQ).transpose(0, 2, 1)
    pe2 = pe[0, :_SEQ, :]
    out5 = fn(idx, pe2, table)
    return out5.transpose(2, 4, 0, 1, 3).reshape(_BATCH, _SEQ, _D)


# speedup vs baseline: 1.5864x; 1.1688x over previous
import functools
import math

import jax
import jax.numpy as jnp
from jax import lax
from jax.experimental import pallas as pl
from jax.experimental.pallas import tpu as pltpu
from jax.experimental.pallas import tpu_sc as plsc

_D = 64
_V = 1000000
_SEQ = 200
_BATCH = 4096
_NC = 2
_NS = 16
_NW = _NC * _NS
_C = _BATCH // _NW
_SCALE = math.sqrt(_D)

_R = 256
_T = _R // 8
_DBLK = _V // _R
_DFULL = _DBLK // _NW
_DEXTRA = _DBLK - _DFULL * _NW
_TAIL_ROWS = _V - _DBLK * _R


def _detile_body(src_hbm, out_hbm, in_v, out_v, sem_i0, sem_i1,
                 sem_o0, sem_o1):
    wid = lax.axis_index("c") * _NS + lax.axis_index("s")
    sem_i = (sem_i0, sem_i1)
    sem_o = (sem_o0, sem_o1)

    def in_slice(blk, n):
        off = pl.multiple_of(blk * _T, 1)
        return src_hbm.at[pl.ds(off, n // 8), :, :]

    def out_slice(blk, n):
        off = pl.multiple_of(blk * (_R // 2), 8)
        return out_hbm.at[pl.ds(off, n // 2), :]

    def start_in(blk, bb, n=_R):
        pltpu.async_copy(in_slice(blk, n), in_v.at[bb, pl.ds(0, n // 8)],
                         sem_i[bb])

    def wait_in(blk, bb, n=_R):
        pltpu.make_async_copy(in_slice(blk, n),
                              in_v.at[bb, pl.ds(0, n // 8)],
                              sem_i[bb]).wait()

    def start_out(blk, bb, n=_R):
        pltpu.async_copy(out_v.at[bb, pl.ds(0, n // 2), :], out_slice(blk, n),
                         sem_o[bb])

    def wait_out(blk, bb, n=_R):
        pltpu.make_async_copy(out_v.at[bb, pl.ds(0, n // 2), :],
                              out_slice(blk, n), sem_o[bb]).wait()

    def compact(bb, n=_R):
        iv = in_v.at[bb]
        ov = out_v.at[bb]

        @plsc.parallel_loop(0, n // 2, step=1, unroll=4)
        def _(k):
            for j in range(8):
                r = 2 * k + (j >= 4)
                ov[k, pl.ds(16 * j, 16)] = iv[
                    lax.shift_right_logical(r, 3), lax.bitwise_and(r, 7),
                    pl.ds(16 * (j % 4), 16)]

    start_in(wid, 0)

    def pair_body(i2, carry):
        for bb in range(2):
            i = 2 * i2 + bb
            blk = wid + _NW * i

            @pl.when(i + 1 < _DFULL)
            def _():
                start_in(wid + _NW * (i + 1), 1 - bb)

            wait_in(blk, bb)

            @pl.when(i2 > 0)
            def _():
                wait_out(wid + _NW * (i - 2), bb)

            compact(bb)
            start_out(blk, bb)
        return carry

    lax.fori_loop(0, _DFULL // 2, pair_body, 0)
    for bb in range(2):
        wait_out(wid + _NW * (_DFULL - 2 + bb), bb)

    @pl.when(wid < _DEXTRA)
    def _():
        blk = _DFULL * _NW + wid
        start_in(blk, 0)
        wait_in(blk, 0)
        compact(0)
        start_out(blk, 0)
        wait_out(blk, 0)

    @pl.when(wid == _DEXTRA)
    def _():
        blk = _DBLK
        start_in(blk, 0, _TAIL_ROWS)
        wait_in(blk, 0, _TAIL_ROWS)
        compact(0, _TAIL_ROWS)
        start_out(blk, 0, _TAIL_ROWS)
        wait_out(blk, 0, _TAIL_ROWS)


def _gather_body(idx_hbm, pe_hbm, table_hbm, out_hbm,
                 idx_v, pe_v, rows_v, trans_v,
                 sem_g0, sem_g1, sem_o0, sem_o1):
    wid = lax.axis_index("c") * _NS + lax.axis_index("s")
    sem_g = (sem_g0, sem_g1)
    sem_o = (sem_o0, sem_o1)

    pltpu.sync_copy(idx_hbm.at[wid], idx_v)
    pltpu.sync_copy(pe_hbm, pe_v)

    lane = lax.iota(jnp.int32, 16)
    dhi = [lax.shift_right_logical(lane + 16 * j, 3) for j in range(_D // 16)]
    dlo = [lax.bitwise_and(lane + 16 * j, 7) for j in range(_D // 16)]

    pltpu.async_copy(table_hbm.at[idx_v.at[0]], rows_v.at[0], sem_g[0])

    def pair_body(s2, carry):
        for b in range(2):
            s = 2 * s2 + b

            @pl.when(s + 1 < _SEQ)
            def _():
                pltpu.async_copy(table_hbm.at[idx_v.at[s + 1]],
                                 rows_v.at[1 - b], sem_g[1 - b])

            pltpu.make_async_copy(table_hbm.at[idx_v.at[s]],
                                  rows_v.at[b], sem_g[b]).wait()

            @pl.when(s2 > 0)
            def _():
                pltpu.make_async_copy(trans_v.at[b, :, :, pl.ds(0, _C)],
                                      out_hbm.at[s - 2, :, wid],
                                      sem_o[b]).wait()

            rv = rows_v.at[b]
            tv = trans_v.at[b]
            pe_s = [pe_v[s, pl.ds(16 * j, 16)] for j in range(_D // 16)]

            @plsc.parallel_loop(0, _C, step=1, unroll=8)
            def _(t):
                col = jnp.full((16,), t, jnp.int32)
                for j in range(_D // 16):
                    v = rv[t, pl.ds(16 * j, 16)] * _SCALE + pe_s[j]
                    plsc.store_scatter(tv, [dhi[j], dlo[j], col], v)

            pltpu.async_copy(tv.at[:, :, pl.ds(0, _C)],
                             out_hbm.at[s, :, wid], sem_o[b])
        return carry

    lax.fori_loop(0, _SEQ // 2, pair_body, 0)
    for b in range(2):
        pltpu.make_async_copy(trans_v.at[b, :, :, pl.ds(0, _C)],
                              out_hbm.at[_SEQ - 2 + b, :, wid],
                              sem_o[b]).wait()


def kernel(x, table, pe):
    mesh = plsc.VectorSubcoreMesh(core_axis_name="c", subcore_axis_name="s")

    detile = functools.partial(
        pl.kernel,
        mesh=mesh,
        out_type=jax.ShapeDtypeStruct((_V // 2, 2 * _D), jnp.float32),
        scratch_types=[
            pltpu.VMEM((2, _T, 8, _D), jnp.float32),
            pltpu.VMEM((2, _R // 2, 2 * _D), jnp.float32),
            pltpu.SemaphoreType.DMA,
            pltpu.SemaphoreType.DMA,
            pltpu.SemaphoreType.DMA,
            pltpu.SemaphoreType.DMA,
        ],
        compiler_params=pltpu.CompilerParams(use_tc_tiling_on_sc=True,
                                             needs_layout_passes=False),
    )(_detile_body)

    gather = functools.partial(
        pl.kernel,
        mesh=mesh,
        out_type=jax.ShapeDtypeStruct((_SEQ, _D // 8, _NW, 8, _C),
                                      jnp.float32),
        scratch_types=[
            pltpu.VMEM((_SEQ, _C), jnp.int32),
            pltpu.VMEM((_SEQ, _D), jnp.float32),
            pltpu.VMEM((2, _C, _D), jnp.float32),
            pltpu.VMEM((2, _D // 8, 8, _C + 1), jnp.float32),
            pltpu.SemaphoreType.DMA,
            pltpu.SemaphoreType.DMA,
            pltpu.SemaphoreType.DMA,
            pltpu.SemaphoreType.DMA,
        ],
        compiler_params=pltpu.CompilerParams(use_tc_tiling_on_sc=False,
                                             needs_layout_passes=False),
    )(_gather_body)

    tbl = detile(table.reshape(_V // 8, 8, _D)).reshape(_V, _D)

    idx = x.astype(jnp.int32).reshape(_NW, _C, _SEQ).transpose(0, 2, 1)
    pe2 = pe[0, :_SEQ, :]
    out5 = gather(idx, pe2, tbl)
    return out5.transpose(2, 4, 0, 1, 3).reshape(_BATCH, _SEQ, _D)
